# baseline (device time: 48280 ns/iter reference)
import jax
import jax.numpy as jnp
from jax import lax
from jax.experimental import pallas as pl
from jax.experimental.pallas import tpu as pltpu

N_DEV = 16
N_ROWS = 1024
D_IN = 256
D_OUT = 512
N_EXP = 64
E_LOCAL = 4
ROWS_PER = N_ROWS // N_DEV


def kernel(x, router_W, route_idx, expert_W, shared_W):
    def body(x_ref, rW_ref, idx_ref, idx_smem, eW_ref, sW_ref, out_ref,
             row_ref, comm_ref, send_sem, recv_sem):
        my = lax.axis_index("i")

        xv = x_ref[...]
        scores = jnp.dot(xv, rW_ref[...],
                         preferred_element_type=jnp.float32)
        m = jnp.max(scores, axis=1, keepdims=True)
        e = jnp.exp(scores - m)
        probs = e / jnp.sum(e, axis=1, keepdims=True)

        sel = idx_ref[...]
        e_iota = lax.broadcasted_iota(jnp.int32, (N_ROWS, N_EXP), 1)
        pv = jnp.sum(jnp.where(e_iota == sel, probs, 0.0),
                     axis=1, keepdims=True)

        xb = xv.astype(jnp.bfloat16)
        partial = jnp.zeros((N_ROWS, D_OUT), jnp.float32)
        for el in range(E_LOCAL):
            eg = E_LOCAL * my + el
            coef = jnp.where(sel == eg, pv, 0.0)
            w = eW_ref[el].astype(jnp.bfloat16)
            y = jnp.dot(xb, w, preferred_element_type=jnp.float32)
            partial = partial + coef * y
        row_ref[...] = partial.astype(jnp.bfloat16)[:, None, :]

        owner = sel // E_LOCAL
        m_send = jnp.sum((owner == my).astype(jnp.int32))

        def row_copy(src_row, dst_slot, dev):
            return pltpu.make_async_remote_copy(
                src_ref=row_ref.at[pl.ds(src_row, 1)],
                dst_ref=comm_ref.at[pl.ds(dst_slot, 1)],
                send_sem=send_sem,
                recv_sem=recv_sem,
                device_id=(dev,),
                device_id_type=pl.DeviceIdType.MESH,
            )

        def send_body(i, c):
            own_i = idx_smem[i, 0] // E_LOCAL
            @pl.when(own_i == my)
            def _():
                row_copy(i, lax.rem(i, ROWS_PER), i // ROWS_PER).start()
            return c
        lax.fori_loop(0, N_ROWS, send_body, 0)

        x_my = x_ref[pl.ds(my * ROWS_PER, ROWS_PER), :]
        shared_my = jnp.dot(x_my.astype(jnp.bfloat16),
                            sW_ref[...].astype(jnp.bfloat16),
                            preferred_element_type=jnp.float32)

        for _ in range(ROWS_PER):
            row_copy(0, 0, my).wait_recv()
        comm_vals = comm_ref[...].reshape(ROWS_PER, D_OUT)
        out_ref[...] = shared_my + comm_vals.astype(jnp.float32)

        lax.fori_loop(0, m_send,
                      lambda i, c: (row_copy(0, 0, my).wait_send(), c)[1], 0)

    return pl.pallas_call(
        body,
        out_shape=jax.ShapeDtypeStruct((ROWS_PER, D_OUT), jnp.float32),
        in_specs=[
            pl.BlockSpec(memory_space=pltpu.VMEM),
            pl.BlockSpec(memory_space=pltpu.VMEM),
            pl.BlockSpec(memory_space=pltpu.VMEM),
            pl.BlockSpec(memory_space=pltpu.SMEM),
            pl.BlockSpec(memory_space=pltpu.VMEM),
            pl.BlockSpec(memory_space=pltpu.VMEM),
        ],
        out_specs=pl.BlockSpec(memory_space=pltpu.VMEM),
        scratch_shapes=[
            pltpu.VMEM((N_ROWS, 1, D_OUT), jnp.bfloat16),
            pltpu.VMEM((ROWS_PER, 1, D_OUT), jnp.bfloat16),
            pltpu.SemaphoreType.DMA,
            pltpu.SemaphoreType.DMA,
        ],
    )(x, router_W, route_idx, route_idx, expert_W, shared_W)
